# baseline (speedup 1.0000x reference)
import functools

import jax
import jax.numpy as jnp
from jax.experimental import pallas as pl
from jax.experimental.pallas import tpu as pltpu

D_MODEL = 4096
N_EXPERTS = 64
TOPK = 8
N_GROUPS = 8
GROUP_SIZE = N_EXPERTS // N_GROUPS
TOPK_GROUPS = 4
ROUTE_SCALE = 2.5

BLOCK_T = 1024
NEG = -jnp.inf


def _gate_body(x_ref, w_ref, b_ref, wout_ref, iout_ref):
    logits = jnp.dot(x_ref[:], w_ref[:], preferred_element_type=jnp.float32)
    lt = logits.T
    orig = jax.nn.sigmoid(lt)
    scores = orig + b_ref[:]
    bT = scores.shape[1]

    sub8 = jax.lax.broadcasted_iota(jnp.int32, (GROUP_SIZE, bT), 0)

    gsums = []
    for g in range(N_GROUPS):
        sg = scores[g * GROUP_SIZE:(g + 1) * GROUP_SIZE, :]
        m1 = jnp.max(sg, axis=0, keepdims=True)
        i1 = jnp.min(jnp.where(sg == m1, sub8, GROUP_SIZE), axis=0, keepdims=True)
        m2 = jnp.max(jnp.where(sub8 == i1, NEG, sg), axis=0, keepdims=True)
        gsums.append(m1 + m2)
    gscore = jnp.concatenate(gsums, axis=0)

    gmask = []
    gwork = gscore
    for _ in range(TOPK_GROUPS):
        m = jnp.max(gwork, axis=0, keepdims=True)
        gi = jnp.min(jnp.where(gwork == m, sub8, N_GROUPS), axis=0, keepdims=True)
        hit = sub8 == gi
        gmask.append(gi)
        gwork = jnp.where(hit, NEG, gwork)

    keep = jnp.zeros((N_GROUPS, bT), dtype=jnp.float32)
    for gi in gmask:
        keep = jnp.where(sub8 == gi, 1.0, keep)
    pieces = [
        scores[g * GROUP_SIZE:(g + 1) * GROUP_SIZE, :] * keep[g:g + 1, :]
        for g in range(N_GROUPS)
    ]
    masked = jnp.concatenate(pieces, axis=0)

    sub64 = jax.lax.broadcasted_iota(jnp.int32, (N_EXPERTS, bT), 0)
    idxs = []
    wvals = []
    work = masked
    for _ in range(TOPK):
        m = jnp.max(work, axis=0, keepdims=True)
        ei = jnp.min(jnp.where(work == m, sub64, N_EXPERTS), axis=0, keepdims=True)
        hit = sub64 == ei
        idxs.append(ei)
        wvals.append(jnp.max(jnp.where(hit, orig, NEG), axis=0, keepdims=True))
        work = jnp.where(hit, NEG, work)
    indices = jnp.concatenate(idxs, axis=0)
    weights = jnp.concatenate(wvals, axis=0)
    weights = weights / jnp.sum(weights, axis=0, keepdims=True) * ROUTE_SCALE

    wout_ref[:] = weights.T
    iout_ref[:] = indices.T


@functools.partial(jax.jit, static_argnames=("interpret",))
def _gate(xf, W, bias_col, interpret=False):
    T = xf.shape[0]
    grid = (T // BLOCK_T,)
    return pl.pallas_call(
        _gate_body,
        grid=grid,
        in_specs=[
            pl.BlockSpec((BLOCK_T, D_MODEL), lambda i: (i, 0)),
            pl.BlockSpec((D_MODEL, N_EXPERTS), lambda i: (0, 0)),
            pl.BlockSpec((N_EXPERTS, 1), lambda i: (0, 0)),
        ],
        out_specs=[
            pl.BlockSpec((BLOCK_T, TOPK), lambda i: (i, 0)),
            pl.BlockSpec((BLOCK_T, TOPK), lambda i: (i, 0)),
        ],
        out_shape=[
            jax.ShapeDtypeStruct((T, TOPK), jnp.float32),
            jax.ShapeDtypeStruct((T, TOPK), jnp.int32),
        ],
        compiler_params=pltpu.CompilerParams(
            dimension_semantics=("arbitrary",),
        ),
        interpret=interpret,
    )(xf, W, bias_col)


def kernel(x, W, bias):
    bsz, seq_len, h = x.shape
    xf = x.reshape(-1, h)
    weights, indices = _gate(xf, W, bias.reshape(N_EXPERTS, 1))
    return weights.astype(x.dtype), indices
